# Initial kernel scaffold; baseline (speedup 1.0000x reference)
import jax
import jax.numpy as jnp
from jax.experimental import pallas as pl


def kernel(cat_idx, table):
    raise NotImplementedError("write your pallas kernel here")


# speedup vs baseline: 1.5829x; 1.5829x over previous
import functools

import jax
import jax.numpy as jnp
from jax import lax
from jax.experimental import pallas as pl
from jax.experimental.pallas import tpu as pltpu
from jax.experimental.pallas import tpu_sc as plsc

_info = plsc.get_sparse_core_info()
_NC = _info.num_cores
_NS = _info.num_subcores
_NW = _NC * _NS


@functools.lru_cache(maxsize=None)
def _make_gather(V, D, B):
    assert B % _NW == 0
    b_per_w = B // _NW
    n_chunks = 8
    assert b_per_w % n_chunks == 0
    chunk = b_per_w // n_chunks
    assert chunk % 8 == 0
    mesh = plsc.VectorSubcoreMesh(core_axis_name="c", subcore_axis_name="s")

    @functools.partial(
        pl.kernel,
        mesh=mesh,
        out_type=jax.ShapeDtypeStruct((B, D), jnp.float32),
        scratch_types=[
            pltpu.VMEM((b_per_w,), jnp.int32),
            pltpu.VMEM((chunk, D), jnp.float32),
            pltpu.VMEM((chunk, D), jnp.float32),
            pltpu.SemaphoreType.DMA,
            pltpu.SemaphoreType.DMA,
            pltpu.SemaphoreType.DMA,
            pltpu.SemaphoreType.DMA,
        ],
        compiler_params=pltpu.CompilerParams(use_tc_tiling_on_sc=False),
    )
    def gather_kernel(table_hbm, idx_hbm, out_hbm,
                      idx_v, rows_a, rows_b, sg_a, sg_b, sw_a, sw_b):
        wid = lax.axis_index("s") * _NC + lax.axis_index("c")
        base = wid * b_per_w

        pltpu.sync_copy(idx_hbm.at[pl.ds(base, b_per_w)], idx_v)

        bufs = (rows_a, rows_b)
        sgs = (sg_a, sg_b)
        sws = (sw_a, sw_b)
        gathers = {}
        writes = {}

        def start_gather(c):
            b = c % 2
            gathers[c] = pltpu.async_copy(
                table_hbm.at[idx_v.at[pl.ds(c * chunk, chunk)]], bufs[b],
                sgs[b])

        def start_write(c):
            b = c % 2
            writes[c] = pltpu.async_copy(
                bufs[b], out_hbm.at[pl.ds(base + c * chunk, chunk)], sws[b])

        start_gather(0)
        start_gather(1)
        for c in range(n_chunks):
            gathers[c].wait()
            start_write(c)
            if c + 2 < n_chunks:
                writes[c].wait()
                start_gather(c + 2)
        writes[n_chunks - 2].wait()
        writes[n_chunks - 1].wait()

    return gather_kernel


def kernel(cat_idx, table):
    batch, fields = cat_idx.shape
    V, D = table.shape
    B = batch * fields
    idx_flat = cat_idx.reshape(B).astype(jnp.int32)
    out = _make_gather(V, D, B)(table, idx_flat)
    return out.reshape(batch, fields, D)
